# baseline (speedup 1.0000x reference)
import functools

import jax
import jax.numpy as jnp
from jax import lax
from jax.experimental import pallas as pl
from jax.experimental.pallas import tpu as pltpu
from jax.experimental.pallas import tpu_sc as plsc

B = 1024
L = 20
N = B * L
D = 64
H = 768
V = 1000000

NW = 32
TPW = N // NW
BIN_SHIFT = 15
BIN_W = 1 << BIN_SHIFT
CM = 512
NBUF = 2
NCHUNK = -(-BIN_W // CM)
V_ALIGNED = (V // 128) * 128
LAST_M0 = V_ALIGNED - CM
TAIL_M0 = V - 128
RB = 128
SENT = 1 << 30

_mesh = functools.partial(
    plsc.VectorSubcoreMesh, core_axis_name="c", subcore_axis_name="s"
)


def _wid():
    return lax.axis_index("s") * 2 + lax.axis_index("c")


@functools.cache
def _make_bin():
    @functools.partial(
        pl.kernel,
        mesh=_mesh(),
        out_type=jax.ShapeDtypeStruct((NW, NW, TPW), jnp.int32),
        scratch_types=[
            pltpu.VMEM((TPW,), jnp.int32),
            pltpu.VMEM((NW, TPW), jnp.int32),
            pltpu.SemaphoreType.DMA,
        ],
        compiler_params=pltpu.CompilerParams(needs_layout_passes=False),
    )
    def bin_kernel(idx_hbm, binned_hbm, idx_v, mat_v, sem):
        w = _wid()
        pltpu.sync_copy(idx_hbm.at[pl.ds(w * TPW, TPW)], idx_v)
        sent_v = jnp.full((16,), SENT, jnp.int32)

        def init_row(r, carry):
            for c in range(TPW // 16):
                mat_v[r, pl.ds(c * 16, 16)] = sent_v
            return carry

        lax.fori_loop(0, NW, init_row, 0)

        iota = lax.iota(jnp.int32, 16)

        def scat(g, carry):
            tok = idx_v[pl.ds(g * 16, 16)]
            bn = lax.shift_right_logical(tok, BIN_SHIFT)
            col = g * 16 + iota
            plsc.store_scatter(mat_v, [bn, col], tok)
            return carry

        lax.fori_loop(0, TPW // 16, scat, 0)

        copies = [
            pltpu.async_copy(mat_v.at[b], binned_hbm.at[b, w], sem)
            for b in range(NW)
        ]
        for cp in copies:
            cp.wait()

    return bin_kernel


@functools.cache
def _make_sweep():
    @functools.partial(
        pl.kernel,
        mesh=_mesh(),
        out_type=jax.ShapeDtypeStruct((N, D), jnp.float32),
        scratch_types=[
            pltpu.VMEM((N + 16,), jnp.int32),
            pltpu.VMEM((N + 16,), jnp.int32),
            pltpu.VMEM((NBUF, D, CM), jnp.float32),
            pltpu.VMEM((RB, D), jnp.float32),
            pltpu.SemaphoreType.DMA,
            pltpu.SemaphoreType.DMA,
            pltpu.SemaphoreType.DMA,
        ],
        compiler_params=pltpu.CompilerParams(needs_layout_passes=False),
    )
    def sweep_kernel(binned_hbm, table_hbm, tail_hbm, out_hbm,
                     cand_v, npos_v, bufs_v, rowbuf_v, sem_in, sem_out, sem_c):
        w = _wid()
        copies = [
            pltpu.async_copy(binned_hbm.at[w, r], cand_v.at[pl.ds(r * TPW, TPW)],
                             sem_c)
            for r in range(NW)
        ]
        for cp in copies:
            cp.wait()

        iota = lax.iota(jnp.int32, 16)
        mb = w * BIN_W

        def cgrp_q(q):
            def cgrp(g, cnt):
                tok = cand_v[pl.ds(g * 16, 16)]
                msk = jnp.logical_and(
                    tok < SENT,
                    lax.shift_right_logical(tok - mb, BIN_SHIFT - 2) == q)
                plsc.store_compressed(npos_v.at[pl.ds(cnt, 16)], g * 16 + iota,
                                      mask=msk)
                return cnt + plsc.all_reduce_population_count(msk)[0]
            return cgrp

        qb0 = lax.fori_loop(0, N // 16, cgrp_q(0), jnp.int32(0))
        qb1 = lax.fori_loop(0, N // 16, cgrp_q(1), qb0)
        qb2 = lax.fori_loop(0, N // 16, cgrp_q(2), qb1)
        cnt = lax.fori_loop(0, N // 16, cgrp_q(3), qb2)
        cand_v[pl.ds(N, 16)] = jnp.full((16,), SENT, jnp.int32)
        npos_v[pl.ds(cnt, 16)] = jnp.full((16,), N, jnp.int32)

        def grp_range(m0):
            q = lax.shift_right_logical(m0 - mb, BIN_SHIFT - 2)
            g_lo = jnp.where(q == 0, 0,
                             jnp.where(q == 1, qb0,
                                       jnp.where(q == 2, qb1, qb2))) // 16
            hi = jnp.where(q == 0, qb0,
                           jnp.where(q == 1, qb1,
                                     jnp.where(q == 2, qb2, cnt)))
            return g_lo, (hi + 15) // 16

        def chunk_m0(k):
            return jnp.minimum(mb + k * CM, LAST_M0)

        def issue(k):
            return pltpu.async_copy(
                table_hbm.at[:, pl.ds(chunk_m0(k), CM)],
                bufs_v.at[lax.rem(k, NBUF)],
                sem_in,
            )

        def drain_out(n_drain):
            def dbody(i, carry):
                pltpu.make_async_copy(
                    out_hbm.at[pl.ds(0, 1)], rowbuf_v.at[pl.ds(0, 1)], sem_out
                ).wait()
                return carry

            lax.fori_loop(0, n_drain, dbody, 0)

        def extract(g, carry, m0, width, parity):
            c_hit, drained = carry
            pos = npos_v[pl.ds(g * 16, 16)]
            tok = plsc.load_gather(cand_v, [pos])
            msk = jnp.logical_and(tok >= m0, tok < m0 + width)
            nh = plsc.all_reduce_population_count(msk)[0]
            new_c = c_hit + nh

            wraps = jnp.logical_and(new_c // RB > c_hit // RB,
                                    c_hit - drained > 0)
            need = jnp.where(wraps, c_hit - drained, 0)
            drain_out(need)

            @pl.when(nh > 0)
            def _():
                off = jnp.where(msk, tok - m0, 0)
                nvec = pos
                mski = msk.astype(jnp.int32)
                slot = lax.rem(c_hit + plsc.cumsum(mski) - 1, RB)
                par_v = jnp.full((16,), parity, jnp.int32)
                for k16 in range(16):
                    @pl.when(mski[k16] != 0)
                    def _():
                        off_v = jnp.full((16,), off[k16], jnp.int32)
                        for j in range(D // 16):
                            vals = plsc.load_gather(
                                bufs_v, [par_v, j * 16 + iota, off_v])
                            rowbuf_v[slot[k16], pl.ds(j * 16, 16)] = vals
                        pltpu.async_copy(
                            rowbuf_v.at[pl.ds(slot[k16], 1)],
                            out_hbm.at[pl.ds(nvec[k16], 1)],
                            sem_out,
                        )

            return new_c, drained + need

        @pl.when(cnt > 0)
        def _():
            for kp in range(NBUF - 1):
                issue(kp)

            def chunk(k, carry):
                @pl.when(k + NBUF - 1 < NCHUNK)
                def _():
                    issue(k + NBUF - 1)

                pltpu.make_async_copy(
                    table_hbm.at[:, pl.ds(chunk_m0(k), CM)],
                    bufs_v.at[lax.rem(k, NBUF)],
                    sem_in,
                ).wait()

                m0 = chunk_m0(k)
                g_lo, g_hi = grp_range(m0)
                body = lambda g, c: extract(g, c, m0, CM, lax.rem(k, NBUF))
                return lax.fori_loop(g_lo, g_hi, body, carry)

            carry = lax.fori_loop(0, NCHUNK, chunk,
                                  (jnp.int32(0), jnp.int32(0)))

            pltpu.sync_copy(tail_hbm, bufs_v.at[0, :, pl.ds(0, 128)])
            g_lo, g_hi = grp_range(jnp.int32(TAIL_M0))
            tbody = lambda g, c: extract(g, c, jnp.int32(TAIL_M0), 128,
                                         jnp.int32(0))
            c_hit, drained = lax.fori_loop(g_lo, g_hi, tbody, carry)

            drain_out(c_hit - drained)

    return sweep_kernel


def _proj_body(x_ref, w_ref, b_ref, o_ref):
    o_ref[...] = (
        lax.dot_general(
            x_ref[...], w_ref[...],
            (((1,), (0,)), ((), ())),
            preferred_element_type=jnp.float32,
        )
        + b_ref[...]
    ).reshape(1, B, H)


def _proj(x, w, b2):
    return pl.pallas_call(
        _proj_body,
        grid=(L,),
        in_specs=[
            pl.BlockSpec((B, D), lambda i: (i, 0)),
            pl.BlockSpec((D, H), lambda i: (0, 0)),
            pl.BlockSpec((1, H), lambda i: (0, 0)),
        ],
        out_specs=pl.BlockSpec((1, B, H), lambda i: (i, 0, 0)),
        out_shape=jax.ShapeDtypeStruct((L, B, H), jnp.float32),
        compiler_params=pltpu.CompilerParams(
            dimension_semantics=("arbitrary",),
        ),
    )(x, w, b2)


def kernel(tokens, table, W, b):
    idx_flat = tokens.T.reshape(N)
    table_t = table.T
    tail = lax.slice(table_t, (0, TAIL_M0), (D, V))
    binned = _make_bin()(idx_flat)
    gathered = _make_sweep()(binned, table_t, tail)
    out_t = _proj(gathered, W.T, b.reshape(1, H))
    return out_t.transpose(1, 0, 2)


# speedup vs baseline: 1.0488x; 1.0488x over previous
import functools

import jax
import jax.numpy as jnp
from jax import lax
from jax.experimental import pallas as pl
from jax.experimental.pallas import tpu as pltpu
from jax.experimental.pallas import tpu_sc as plsc

B = 1024
L = 20
N = B * L
D = 64
H = 768
V = 1000000

NW = 32
TPW = N // NW
BIN_SHIFT = 15
BIN_W = 1 << BIN_SHIFT
CM = 512
NBUF = 2
NCHUNK = -(-BIN_W // CM)
V_ALIGNED = (V // 128) * 128
LAST_M0 = V_ALIGNED - CM
TAIL_M0 = V - 128
RB = 128
SENT = 1 << 30

_mesh = functools.partial(
    plsc.VectorSubcoreMesh, core_axis_name="c", subcore_axis_name="s"
)


def _wid():
    return lax.axis_index("s") * 2 + lax.axis_index("c")


@functools.cache
def _make_bin():
    @functools.partial(
        pl.kernel,
        mesh=_mesh(),
        out_type=jax.ShapeDtypeStruct((NW, NW, TPW), jnp.int32),
        scratch_types=[
            pltpu.VMEM((TPW,), jnp.int32),
            pltpu.VMEM((NW, TPW), jnp.int32),
            pltpu.SemaphoreType.DMA,
        ],
        compiler_params=pltpu.CompilerParams(needs_layout_passes=False),
    )
    def bin_kernel(idx_hbm, binned_hbm, idx_v, mat_v, sem):
        w = _wid()
        pltpu.sync_copy(idx_hbm.at[pl.ds(w * TPW, TPW)], idx_v)
        sent_v = jnp.full((16,), SENT, jnp.int32)

        def init_row(r, carry):
            for c in range(TPW // 16):
                mat_v[r, pl.ds(c * 16, 16)] = sent_v
            return carry

        lax.fori_loop(0, NW, init_row, 0)

        iota = lax.iota(jnp.int32, 16)

        def scat(g, carry):
            tok = idx_v[pl.ds(g * 16, 16)]
            bn = lax.shift_right_logical(tok, BIN_SHIFT)
            col = g * 16 + iota
            plsc.store_scatter(mat_v, [bn, col], tok)
            return carry

        lax.fori_loop(0, TPW // 16, scat, 0)

        copies = [
            pltpu.async_copy(mat_v.at[b], binned_hbm.at[b, w], sem)
            for b in range(NW)
        ]
        for cp in copies:
            cp.wait()

    return bin_kernel


@functools.cache
def _make_sweep():
    @functools.partial(
        pl.kernel,
        mesh=_mesh(),
        out_type=jax.ShapeDtypeStruct((N, D), jnp.float32),
        scratch_types=[
            pltpu.VMEM((N + 16,), jnp.int32),
            pltpu.VMEM((N + 16,), jnp.int32),
            pltpu.VMEM((NBUF, D // 8, 8, CM), jnp.float32),
            pltpu.VMEM((RB, D), jnp.float32),
            pltpu.SemaphoreType.DMA,
            pltpu.SemaphoreType.DMA,
            pltpu.SemaphoreType.DMA,
        ],
        compiler_params=pltpu.CompilerParams(needs_layout_passes=False),
    )
    def sweep_kernel(binned_hbm, table_hbm, tail_hbm, out_hbm,
                     cand_v, npos_v, bufs_v, rowbuf_v, sem_in, sem_out, sem_c):
        w = _wid()
        copies = [
            pltpu.async_copy(binned_hbm.at[w, r], cand_v.at[pl.ds(r * TPW, TPW)],
                             sem_c)
            for r in range(NW)
        ]
        for cp in copies:
            cp.wait()

        iota = lax.iota(jnp.int32, 16)
        mb = w * BIN_W

        def cgrp_q(q):
            def cgrp(g, cnt):
                tok = cand_v[pl.ds(g * 16, 16)]
                msk = jnp.logical_and(
                    tok < SENT,
                    lax.shift_right_logical(tok - mb, BIN_SHIFT - 2) == q)
                plsc.store_compressed(npos_v.at[pl.ds(cnt, 16)], g * 16 + iota,
                                      mask=msk)
                return cnt + plsc.all_reduce_population_count(msk)[0]
            return cgrp

        qb0 = lax.fori_loop(0, N // 16, cgrp_q(0), jnp.int32(0))
        qb1 = lax.fori_loop(0, N // 16, cgrp_q(1), qb0)
        qb2 = lax.fori_loop(0, N // 16, cgrp_q(2), qb1)
        cnt = lax.fori_loop(0, N // 16, cgrp_q(3), qb2)
        cand_v[pl.ds(N, 16)] = jnp.full((16,), SENT, jnp.int32)
        npos_v[pl.ds(cnt, 16)] = jnp.full((16,), N, jnp.int32)

        def grp_range(m0):
            q = lax.shift_right_logical(m0 - mb, BIN_SHIFT - 2)
            g_lo = jnp.where(q == 0, 0,
                             jnp.where(q == 1, qb0,
                                       jnp.where(q == 2, qb1, qb2))) // 16
            hi = jnp.where(q == 0, qb0,
                           jnp.where(q == 1, qb1,
                                     jnp.where(q == 2, qb2, cnt)))
            return g_lo, (hi + 15) // 16

        def chunk_m0(k):
            return jnp.minimum(mb + k * CM, LAST_M0)

        def issue(k):
            for dt in range(D // 8):
                pltpu.async_copy(
                    table_hbm.at[pl.ds(dt * 8, 8), pl.ds(chunk_m0(k), CM)],
                    bufs_v.at[lax.rem(k, NBUF), dt],
                    sem_in,
                )

        def drain_out(n_drain):
            def dbody(i, carry):
                pltpu.make_async_copy(
                    out_hbm.at[pl.ds(0, 1)], rowbuf_v.at[pl.ds(0, 1)], sem_out
                ).wait()
                return carry

            lax.fori_loop(0, n_drain, dbody, 0)

        def extract(g, carry, m0, width, parity):
            c_hit, drained = carry
            pos = npos_v[pl.ds(g * 16, 16)]
            tok = plsc.load_gather(cand_v, [pos])
            msk = jnp.logical_and(tok >= m0, tok < m0 + width)
            nh = plsc.all_reduce_population_count(msk)[0]
            new_c = c_hit + nh

            wraps = jnp.logical_and(new_c // RB > c_hit // RB,
                                    c_hit - drained > 0)
            need = jnp.where(wraps, c_hit - drained, 0)
            drain_out(need)

            @pl.when(nh > 0)
            def _():
                off = jnp.where(msk, tok - m0, 0)
                nvec = pos
                mski = msk.astype(jnp.int32)
                slot = lax.rem(c_hit + plsc.cumsum(mski) - 1, RB)
                par_v = jnp.full((16,), parity, jnp.int32)
                for d in range(D):
                    vals = plsc.load_gather(
                        bufs_v, [par_v, jnp.full((16,), d // 8, jnp.int32),
                                 jnp.full((16,), d % 8, jnp.int32), off],
                        mask=msk)
                    plsc.store_scatter(
                        rowbuf_v, [slot, jnp.full((16,), d, jnp.int32)], vals,
                        mask=msk)
                for k16 in range(16):
                    @pl.when(mski[k16] != 0)
                    def _():
                        pltpu.async_copy(
                            rowbuf_v.at[pl.ds(slot[k16], 1)],
                            out_hbm.at[pl.ds(nvec[k16], 1)],
                            sem_out,
                        )

            return new_c, drained + need

        @pl.when(cnt > 0)
        def _():
            for kp in range(NBUF - 1):
                issue(kp)

            def chunk(k, carry):
                @pl.when(k + NBUF - 1 < NCHUNK)
                def _():
                    issue(k + NBUF - 1)

                for dt in range(D // 8):
                    pltpu.make_async_copy(
                        table_hbm.at[pl.ds(dt * 8, 8), pl.ds(chunk_m0(k), CM)],
                        bufs_v.at[lax.rem(k, NBUF), dt],
                        sem_in,
                    ).wait()

                m0 = chunk_m0(k)
                g_lo, g_hi = grp_range(m0)
                body = lambda g, c: extract(g, c, m0, CM, lax.rem(k, NBUF))
                return lax.fori_loop(g_lo, g_hi, body, carry)

            carry = lax.fori_loop(0, NCHUNK, chunk,
                                  (jnp.int32(0), jnp.int32(0)))

            pltpu.sync_copy(tail_hbm, bufs_v.at[0, :, :, pl.ds(0, 128)])
            g_lo, g_hi = grp_range(jnp.int32(TAIL_M0))
            tbody = lambda g, c: extract(g, c, jnp.int32(TAIL_M0), 128,
                                         jnp.int32(0))
            c_hit, drained = lax.fori_loop(g_lo, g_hi, tbody, carry)

            drain_out(c_hit - drained)

    return sweep_kernel


def _proj_body(x_ref, w_ref, b_ref, o_ref):
    o_ref[...] = (
        lax.dot_general(
            x_ref[...], w_ref[...],
            (((1,), (0,)), ((), ())),
            preferred_element_type=jnp.float32,
        )
        + b_ref[...]
    ).reshape(1, B, H)


def _proj(x, w, b2):
    return pl.pallas_call(
        _proj_body,
        grid=(L,),
        in_specs=[
            pl.BlockSpec((B, D), lambda i: (i, 0)),
            pl.BlockSpec((D, H), lambda i: (0, 0)),
            pl.BlockSpec((1, H), lambda i: (0, 0)),
        ],
        out_specs=pl.BlockSpec((1, B, H), lambda i: (i, 0, 0)),
        out_shape=jax.ShapeDtypeStruct((L, B, H), jnp.float32),
        compiler_params=pltpu.CompilerParams(
            dimension_semantics=("arbitrary",),
        ),
    )(x, w, b2)


def kernel(tokens, table, W, b):
    idx_flat = tokens.T.reshape(N)
    table_t = table.T
    tail = lax.slice(table_t, (0, TAIL_M0), (D, V)).reshape(D // 8, 8, 128)
    binned = _make_bin()(idx_flat)
    gathered = _make_sweep()(binned, table_t, tail)
    out_t = _proj(gathered, W.T, b.reshape(1, H))
    return out_t.transpose(1, 0, 2)
